# Initial kernel scaffold; baseline (speedup 1.0000x reference)
import jax
import jax.numpy as jnp
from jax.experimental import pallas as pl


def kernel(v):
    raise NotImplementedError("write your pallas kernel here")


# speedup vs baseline: 1.9342x; 1.9342x over previous
import functools

import jax
import jax.numpy as jnp
from jax import lax
from jax.experimental import pallas as pl
from jax.experimental.pallas import tpu as pltpu
from jax.experimental.pallas import tpu_sc as plsc

ROWS = 64
N = 32768
L = 16
NV = N // L
RADIX = 256
NPASS = 4
CWORDS = RADIX * L
MININT = -2147483648

_info = plsc.get_sparse_core_info()
NC = _info.num_cores
NS = _info.num_subcores
NW = NC * NS
ROWS_PER_W = ROWS // NW


def _lane():
    return lax.iota(jnp.int32, L)


def _to_sortable(bits):
    s = lax.shift_right_arithmetic(bits, 31)
    return bits ^ (s | MININT)


def _from_sortable(u):
    s = lax.shift_right_arithmetic(u, 31)
    return u ^ (jnp.invert(s) | MININT)


def _sort_kernel(v_hbm, vals_hbm, idx_hbm, keys, idxf, idxi, counter):
    wid = lax.axis_index("s") * NC + lax.axis_index("c")
    lane = _lane()
    ones = jnp.ones((L,), jnp.int32)

    def load_idx(ref, is_f32, i):
        x = ref[pl.ds(i * L, L)]
        return plsc.bitcast(x, jnp.int32) if is_f32 else x

    def gather_key(idx):
        return plsc.bitcast(plsc.load_gather(keys, [idx]), jnp.int32)

    def run_pass(src, src_f32, dst, dst_f32, shift, last):
        def zero(i, c):
            counter[pl.ds(i * L, L)] = jnp.zeros((L,), jnp.int32)
            return c
        lax.fori_loop(0, RADIX, zero, 0)

        def hist(i, c):
            idx = load_idx(src, src_f32, i)
            k = gather_key(idx)
            d = lax.shift_right_logical(k, shift) & (RADIX - 1)
            cidx = lax.shift_left(d, 4) | lane
            plsc.addupdate_scatter(counter, [cidx], ones)
            return c
        lax.fori_loop(0, NV, hist, 0)

        def scan(i, carry):
            c = counter[pl.ds(i * L, L)]
            inc = plsc.cumsum(c)
            counter[pl.ds(i * L, L)] = inc - c + carry
            return carry + jnp.sum(c)
        lax.fori_loop(0, RADIX, scan, jnp.int32(0))

        def perm(i, c):
            idx = load_idx(src, src_f32, i)
            k = gather_key(idx)
            d = lax.shift_right_logical(k, shift) & (RADIX - 1)
            cidx = lax.shift_left(d, 4) | lane
            pos = plsc.load_gather(counter, [cidx])
            plsc.store_scatter(counter, [cidx], pos + ones)
            if last:
                addr = pos
            else:
                addr = lax.shift_left(pos & (NV - 1), 4) | \
                    lax.shift_right_logical(pos, 11)
            out = plsc.bitcast(idx, jnp.float32) if dst_f32 else idx
            plsc.store_scatter(dst, [addr], out)
            return c
        lax.fori_loop(0, NV, perm, 0)

    for r in range(ROWS_PER_W):
        row = wid * ROWS_PER_W + r
        pltpu.sync_copy(v_hbm.at[row], keys)

        def xform(i, c):
            b = plsc.bitcast(keys[pl.ds(i * L, L)], jnp.int32)
            keys[pl.ds(i * L, L)] = plsc.bitcast(_to_sortable(b), jnp.float32)
            return c
        lax.fori_loop(0, NV, xform, 0)

        def fill(i, c):
            idxi[pl.ds(i * L, L)] = lane * NV + i
            return c
        lax.fori_loop(0, NV, fill, 0)

        run_pass(idxi, False, idxf, True, 0, False)
        run_pass(idxf, True, idxi, False, 8, False)
        run_pass(idxi, False, idxf, True, 16, False)
        run_pass(idxf, True, idxi, False, 24, True)

        def vals(i, c):
            idx = idxi[pl.ds(i * L, L)]
            u = gather_key(idx)
            idxf[pl.ds(i * L, L)] = plsc.bitcast(_from_sortable(u),
                                                 jnp.float32)
            return c
        lax.fori_loop(0, NV, vals, 0)

        pltpu.sync_copy(idxf, vals_hbm.at[row])
        pltpu.sync_copy(idxi, idx_hbm.at[row])


@jax.jit
def kernel(v):
    mesh = plsc.VectorSubcoreMesh(core_axis_name="c", subcore_axis_name="s")
    f = pl.kernel(
        _sort_kernel,
        out_type=(
            jax.ShapeDtypeStruct((ROWS, N), jnp.float32),
            jax.ShapeDtypeStruct((ROWS, N), jnp.int32),
        ),
        mesh=mesh,
        scratch_types=[
            pltpu.VMEM((N,), jnp.float32),
            pltpu.VMEM((N,), jnp.float32),
            pltpu.VMEM((N,), jnp.int32),
            pltpu.VMEM((CWORDS,), jnp.int32),
        ],
        compiler_params=pltpu.CompilerParams(needs_layout_passes=False),
    )
    return f(v)


# speedup vs baseline: 2.0602x; 1.0651x over previous
import jax
import jax.numpy as jnp
from jax import lax
from jax.experimental import pallas as pl
from jax.experimental.pallas import tpu as pltpu
from jax.experimental.pallas import tpu_sc as plsc

ROWS = 64
N = 32768
L = 16
NV = N // L
RADIX = 256
CWORDS = RADIX * L
MININT = -2147483648

_info = plsc.get_sparse_core_info()
NC = _info.num_cores
NS = _info.num_subcores
NW = NC * NS
ROWS_PER_W = ROWS // NW


def _to_sortable(bits):
    s = lax.shift_right_arithmetic(bits, 31)
    return bits ^ (s | MININT)


def _from_sortable(u):
    s = lax.shift_right_arithmetic(u, 31)
    return u ^ (jnp.invert(s) | MININT)


def _sort_kernel(v_hbm, vals_hbm, idx_hbm, keys, idxf, idxi, counter):
    wid = lax.axis_index("s") * NC + lax.axis_index("c")
    lane = lax.iota(jnp.int32, L)
    ones = jnp.ones((L,), jnp.int32)

    def load_idx(ref, is_f32, i):
        x = ref[pl.ds(i * L, L)]
        return plsc.bitcast(x, jnp.int32) if is_f32 else x

    def gather_key(idx):
        return plsc.bitcast(plsc.load_gather(keys, [idx]), jnp.int32)

    def zero_and_scan(run_hist):
        def zero(i, c):
            counter[pl.ds(i * L, L)] = jnp.zeros((L,), jnp.int32)
            return c
        lax.fori_loop(0, RADIX, zero, 0, unroll=8)

        lax.fori_loop(0, NV, run_hist, 0, unroll=4)

        def scan(i, carry):
            c = counter[pl.ds(i * L, L)]
            inc = plsc.cumsum(c)
            counter[pl.ds(i * L, L)] = inc - c + carry
            return carry + jnp.sum(c)
        lax.fori_loop(0, RADIX, scan, jnp.int32(0), unroll=4)

    def digit_cidx(k, shift):
        d = lax.shift_right_logical(k, shift) & (RADIX - 1)
        return lax.shift_left(d, 4) | lane

    def rank(cidx):
        pos = plsc.load_gather(counter, [cidx])
        plsc.store_scatter(counter, [cidx], pos + ones)
        return pos

    def col_addr(pos):
        return lax.shift_left(pos & (NV - 1), 4) | \
            lax.shift_right_logical(pos, 11)

    def run_pass(src, src_f32, dst, dst_f32, shift, last):
        def hist(i, c):
            k = gather_key(load_idx(src, src_f32, i))
            plsc.addupdate_scatter(counter, [digit_cidx(k, shift)], ones)
            return c
        zero_and_scan(hist)

        def perm(i, c):
            idx = load_idx(src, src_f32, i)
            k = gather_key(idx)
            pos = rank(digit_cidx(k, shift))
            addr = pos if last else col_addr(pos)
            out = plsc.bitcast(idx, jnp.float32) if dst_f32 else idx
            plsc.store_scatter(dst, [addr], out)
            return c
        lax.fori_loop(0, NV, perm, 0, unroll=4)

    for r in range(ROWS_PER_W):
        row = wid * ROWS_PER_W + r
        pltpu.sync_copy(v_hbm.at[row], keys)

        def fill(i, c):
            b = plsc.bitcast(keys[pl.ds(i * L, L)], jnp.int32)
            keys[pl.ds(i * L, L)] = plsc.bitcast(_to_sortable(b),
                                                 jnp.float32)
            idxi[pl.ds(i * L, L)] = lane * NV + i
            return c
        lax.fori_loop(0, NV, fill, 0, unroll=4)

        run_pass(idxi, False, idxf, True, 0, False)
        run_pass(idxf, True, idxi, False, 8, False)
        run_pass(idxi, False, idxf, True, 16, False)
        run_pass(idxf, True, idxi, False, 24, True)

        def vals(i, c):
            u = gather_key(idxi[pl.ds(i * L, L)])
            idxf[pl.ds(i * L, L)] = plsc.bitcast(_from_sortable(u),
                                                 jnp.float32)
            return c
        lax.fori_loop(0, NV, vals, 0, unroll=4)

        pltpu.sync_copy(idxf, vals_hbm.at[row])
        pltpu.sync_copy(idxi, idx_hbm.at[row])


@jax.jit
def kernel(v):
    mesh = plsc.VectorSubcoreMesh(core_axis_name="c", subcore_axis_name="s")
    f = pl.kernel(
        _sort_kernel,
        out_type=(
            jax.ShapeDtypeStruct((ROWS, N), jnp.float32),
            jax.ShapeDtypeStruct((ROWS, N), jnp.int32),
        ),
        mesh=mesh,
        scratch_types=[
            pltpu.VMEM((N,), jnp.float32),
            pltpu.VMEM((N,), jnp.float32),
            pltpu.VMEM((N,), jnp.int32),
            pltpu.VMEM((CWORDS,), jnp.int32),
        ],
        compiler_params=pltpu.CompilerParams(needs_layout_passes=False),
    )
    return f(v)


# speedup vs baseline: 3.0142x; 1.4630x over previous
import jax
import jax.numpy as jnp
from jax import lax
from jax.experimental import pallas as pl
from jax.experimental.pallas import tpu as pltpu
from jax.experimental.pallas import tpu_sc as plsc

ROWS = 64
N = 32768
L = 16
NV = N // L
RADIX = 256
CWORDS = RADIX * L
MININT = -2147483648
IDXMASK = N - 1

_info = plsc.get_sparse_core_info()
NC = _info.num_cores
NS = _info.num_subcores
NW = NC * NS
ROWS_PER_W = ROWS // NW


def _to_sortable(bits):
    s = lax.shift_right_arithmetic(bits, 31)
    return bits ^ (s | MININT)


def _from_sortable(u):
    s = lax.shift_right_arithmetic(u, 31)
    return u ^ (jnp.invert(s) | MININT)


def _sort_kernel(v_hbm, vals_hbm, idx_hbm, keys, idxf, idxi, counter):
    wid = lax.axis_index("s") * NC + lax.axis_index("c")
    lane = lax.iota(jnp.int32, L)
    ones = jnp.ones((L,), jnp.int32)

    def load_payload(ref, is_f32, i):
        x = ref[pl.ds(i * L, L)]
        return plsc.bitcast(x, jnp.int32) if is_f32 else x

    def gather_key(idx):
        return plsc.bitcast(plsc.load_gather(keys, [idx]), jnp.int32)

    def cidx_of(payload):
        d = lax.shift_right_logical(payload, 15) & (RADIX - 1)
        return lax.shift_left(d, 4) | lane

    def rank(cidx):
        pos = plsc.load_gather(counter, [cidx])
        plsc.store_scatter(counter, [cidx], pos + ones)
        return pos

    def col_addr(pos):
        return lax.shift_left(pos & (NV - 1), 4) | \
            lax.shift_right_logical(pos, 11)

    def zero_and_scan(src, src_f32):
        def zero(i, c):
            counter[pl.ds(i * L, L)] = jnp.zeros((L,), jnp.int32)
            return c
        lax.fori_loop(0, RADIX, zero, 0, unroll=8)

        def hist(i, c):
            plsc.addupdate_scatter(
                counter, [cidx_of(load_payload(src, src_f32, i))], ones)
            return c
        lax.fori_loop(0, NV, hist, 0, unroll=4)

        def scan(i, carry):
            c = counter[pl.ds(i * L, L)]
            inc = plsc.cumsum(c)
            counter[pl.ds(i * L, L)] = inc - c + carry
            return carry + jnp.sum(c)
        lax.fori_loop(0, RADIX, scan, jnp.int32(0), unroll=4)

    def run_pass(src, src_f32, dst, dst_f32, refill, last):
        zero_and_scan(src, src_f32)

        def perm(i, c):
            payload = load_payload(src, src_f32, i)
            pos = rank(cidx_of(payload))
            idx = payload & IDXMASK
            if last:
                plsc.store_scatter(dst, [pos], idx)
            else:
                if refill:
                    k = gather_key(idx)
                    hi = lax.shift_left(
                        lax.shift_right_logical(k, 16), 15)
                else:
                    hi = lax.shift_left(
                        lax.shift_right_logical(payload, 23), 15)
                out = hi | idx
                if dst_f32:
                    out = plsc.bitcast(out, jnp.float32)
                plsc.store_scatter(dst, [col_addr(pos)], out)
            return c
        lax.fori_loop(0, NV, perm, 0, unroll=4)

    def do_row(r, c):
        row = wid * ROWS_PER_W + r
        pltpu.sync_copy(v_hbm.at[row], keys)

        def fill(i, c2):
            b = plsc.bitcast(keys[pl.ds(i * L, L)], jnp.int32)
            k = _to_sortable(b)
            keys[pl.ds(i * L, L)] = plsc.bitcast(k, jnp.float32)
            j = i * L + lane
            payload = lax.shift_left(k & 0xFFFF, 15) | j
            addr = lax.shift_left(j & (NV - 1), 4) | \
                lax.shift_right_logical(j, 11)
            plsc.store_scatter(idxi, [addr], payload)
            return c2
        lax.fori_loop(0, NV, fill, 0, unroll=4)

        run_pass(idxi, False, idxf, True, refill=False, last=False)
        run_pass(idxf, True, idxi, False, refill=True, last=False)
        run_pass(idxi, False, idxf, True, refill=False, last=False)
        run_pass(idxf, True, idxi, False, refill=False, last=True)

        def vals(i, c2):
            u = gather_key(idxi[pl.ds(i * L, L)])
            idxf[pl.ds(i * L, L)] = plsc.bitcast(_from_sortable(u),
                                                 jnp.float32)
            return c2
        lax.fori_loop(0, NV, vals, 0, unroll=4)

        pltpu.sync_copy(idxf, vals_hbm.at[row])
        pltpu.sync_copy(idxi, idx_hbm.at[row])
        return c

    lax.fori_loop(0, ROWS_PER_W, do_row, 0)


@jax.jit
def kernel(v):
    mesh = plsc.VectorSubcoreMesh(core_axis_name="c", subcore_axis_name="s")
    f = pl.kernel(
        _sort_kernel,
        out_type=(
            jax.ShapeDtypeStruct((ROWS, N), jnp.float32),
            jax.ShapeDtypeStruct((ROWS, N), jnp.int32),
        ),
        mesh=mesh,
        scratch_types=[
            pltpu.VMEM((N,), jnp.float32),
            pltpu.VMEM((N,), jnp.float32),
            pltpu.VMEM((N,), jnp.int32),
            pltpu.VMEM((CWORDS,), jnp.int32),
        ],
        compiler_params=pltpu.CompilerParams(needs_layout_passes=False),
    )
    return f(v)
